# baseline (speedup 1.0000x reference)
import functools

import jax
import jax.numpy as jnp
from jax import lax
from jax.experimental import pallas as pl
from jax.experimental.pallas import tpu as pltpu
from jax.experimental.pallas import tpu_sc as plsc

B = 320000
D = 128
N = 10000
W = 64
NC, NS, L = 2, 16, 16
NW = NC * NS
FW = W // NS

NITER = 75
E = 64
Y_W = NITER * E
Y = Y_W * NW
CF = 1280
RW = CF // NS
X_SC = (B - Y) // NC
NF = X_SC // CF
N_STAGE = N // NS


def _edge_dot_kernel(zp_hbm, zpt_hbm, src_hbm, dst_hbm, out_hbm,
                     zsh, board_a, board_b, zcol, sidx_v, didx_v,
                     sbuf_a, dbuf_a, sbuf_b, dbuf_b, out_v,
                     fsidx_a, fdidx_a, fsidx_b, fdidx_b,
                     part_v, rbuf, obuf_a, obuf_b,
                     sem_z, sem_a, sem_b, sem_fa, sem_fb, sem_oa, sem_ob):
    c = lax.axis_index("c")
    s = lax.axis_index("s")
    wid = s * NC + c
    sbase = wid * Y_W
    fbase = Y + c * X_SC
    lane = lax.iota(jnp.int32, L)
    m15 = lane == 15
    rslice = pl.multiple_of(s * RW, 8)

    pltpu.async_copy(src_hbm.at[pl.ds(sbase, Y_W)], sidx_v, sem_z)
    pltpu.async_copy(dst_hbm.at[pl.ds(sbase, Y_W)], didx_v, sem_z)
    pltpu.async_copy(zpt_hbm.at[pl.ds(s * (FW * N), FW * N)], zcol, sem_z)
    pltpu.sync_copy(zp_hbm.at[pl.ds(s * N_STAGE, N_STAGE)],
                    zsh.at[pl.ds(s * N_STAGE, N_STAGE)])
    pltpu.make_async_copy(src_hbm.at[pl.ds(0, Y_W)], sidx_v, sem_z).wait()
    pltpu.make_async_copy(src_hbm.at[pl.ds(0, Y_W)], didx_v, sem_z).wait()
    pltpu.make_async_copy(zpt_hbm.at[pl.ds(0, FW * N)], zcol, sem_z).wait()
    plsc.subcore_barrier()

    def issue(k, sbuf, dbuf, sem):
        pltpu.async_copy(zsh.at[sidx_v.at[pl.ds(k * E, E)]], sbuf, sem)
        pltpu.async_copy(zsh.at[didx_v.at[pl.ds(k * E, E)]], dbuf, sem)

    def wait(sbuf, dbuf, sem):
        pltpu.make_async_copy(zsh.at[pl.ds(0, E)], sbuf, sem).wait()
        pltpu.make_async_copy(zsh.at[pl.ds(0, E)], dbuf, sem).wait()

    cols = [lane + (16 * q) for q in range(W // L)]
    rowzero = jnp.zeros((L,), jnp.int32)

    def compute_stream(k, sbuf, dbuf):
        @plsc.parallel_loop(0, E, 1, unroll=2)
        def _(e):
            base = jnp.full((L,), e * W, jnp.int32)
            acc = None
            for q in range(W // L):
                idx = base + cols[q]
                sv = plsc.bitcast(plsc.load_gather(sbuf, [rowzero, idx]),
                                  jnp.bfloat16)
                dv = plsc.bitcast(plsc.load_gather(dbuf, [rowzero, idx]),
                                  jnp.bfloat16)
                p = sv * dv
                acc = p if acc is None else acc + p
            lo, hi = plsc.unpack(acc, format=plsc.PackFormat.INTERLEAVED)
            tot = plsc.cumsum(lo.astype(jnp.float32) + hi.astype(jnp.float32))
            plsc.store_scatter(out_v, [jnp.full((L,), k * E, jnp.int32) + e],
                               tot, mask=m15)

    offs = [jnp.full((L,), j * N, jnp.int32) for j in range(FW)]

    def issue_fidx(k, fsidx, fdidx, sem):
        pltpu.async_copy(src_hbm.at[pl.ds(fbase + k * CF, CF)], fsidx, sem)
        pltpu.async_copy(dst_hbm.at[pl.ds(fbase + k * CF, CF)], fdidx, sem)

    def wait_fidx(fsidx, fdidx, sem):
        pltpu.make_async_copy(src_hbm.at[pl.ds(0, CF)], fsidx, sem).wait()
        pltpu.make_async_copy(src_hbm.at[pl.ds(0, CF)], fdidx, sem).wait()

    def compute_feature(fsidx, fdidx, board):
        @plsc.parallel_loop(0, CF, L, unroll=2)
        def _(g):
            sn = fsidx[pl.ds(g, L)]
            dn = fdidx[pl.ds(g, L)]
            facc = None
            for j in range(FW):
                sv = plsc.bitcast(plsc.load_gather(zcol, [sn + offs[j]]),
                                  jnp.bfloat16)
                dv = plsc.bitcast(plsc.load_gather(zcol, [dn + offs[j]]),
                                  jnp.bfloat16)
                p = sv * dv
                facc = p if facc is None else facc + p
            lo, hi = plsc.unpack(facc, format=plsc.PackFormat.INTERLEAVED)
            part_v[pl.ds(g, L)] = lo.astype(jnp.float32) + hi.astype(jnp.float32)

        pltpu.sync_copy(part_v, board.at[s])

    def reduce_board(k, board, obuf, sem):
        @pl.when(k >= 2)
        def _():
            pltpu.make_async_copy(obuf, out_hbm.at[pl.ds(0, RW)], sem).wait()

        pltpu.sync_copy(board.at[:, pl.ds(rslice, RW)], rbuf)
        for q in range(RW // L):
            acc = None
            for r in range(NS):
                v = rbuf[r, pl.ds(q * L, L)]
                acc = v if acc is None else acc + v
            obuf[pl.ds(q * L, L)] = acc
        pltpu.async_copy(obuf, out_hbm.at[pl.ds(fbase + k * CF + rslice, RW)],
                         sem)

    issue(0, sbuf_a, dbuf_a, sem_a)
    issue(1, sbuf_b, dbuf_b, sem_b)
    issue_fidx(0, fsidx_a, fdidx_a, sem_fa)
    issue_fidx(1, fsidx_b, fdidx_b, sem_fb)

    def iter_body(k, even, sbuf, dbuf, sem, fsidx, fdidx, sem_f,
                  board, obuf, sem_ob):
        wait(sbuf, dbuf, sem)
        compute_stream(k, sbuf, dbuf)

        @pl.when(k < NITER - 2)
        def _():
            issue(k + 2, sbuf, dbuf, sem)

        @pl.when(k < NF)
        def _():
            wait_fidx(fsidx, fdidx, sem_f)
            compute_feature(fsidx, fdidx, board)

            @pl.when(k < NF - 2)
            def _():
                issue_fidx(k + 2, fsidx, fdidx, sem_f)

            plsc.subcore_barrier()
            reduce_board(k, board, obuf, sem_ob)

    def pair_body(p, _):
        ka = 2 * p
        iter_body(ka, True, sbuf_a, dbuf_a, sem_a, fsidx_a, fdidx_a, sem_fa,
                  board_a, obuf_a, sem_oa)
        iter_body(ka + 1, False, sbuf_b, dbuf_b, sem_b, fsidx_b, fdidx_b,
                  sem_fb, board_b, obuf_b, sem_ob)
        return 0

    lax.fori_loop(0, NITER // 2, pair_body, 0)
    wait(sbuf_a, dbuf_a, sem_a)
    compute_stream(NITER - 1, sbuf_a, dbuf_a)
    pltpu.make_async_copy(obuf_a, out_hbm.at[pl.ds(0, RW)], sem_oa).wait()
    pltpu.make_async_copy(obuf_b, out_hbm.at[pl.ds(0, RW)], sem_ob).wait()

    pltpu.sync_copy(out_v, out_hbm.at[pl.ds(sbase, Y_W)])


@jax.jit
def kernel(z, edge_index):
    zb = z.astype(jnp.bfloat16)
    zp = lax.bitcast_convert_type(zb.reshape(N, W, 2), jnp.int32)
    zpt = zp.T.reshape(W * N)
    src = edge_index[0].astype(jnp.int32)
    dst = edge_index[1].astype(jnp.int32)
    mesh = plsc.VectorSubcoreMesh(core_axis_name="c", subcore_axis_name="s")
    run = functools.partial(
        pl.kernel,
        mesh=mesh,
        out_type=jax.ShapeDtypeStruct((B,), jnp.float32),
        compiler_params=pltpu.CompilerParams(
            use_tc_tiling_on_sc=False, needs_layout_passes=False),
        scratch_types=[
            pltpu.VMEM_SHARED((N, W), jnp.int32),
            pltpu.VMEM_SHARED((NS, CF), jnp.float32),
            pltpu.VMEM_SHARED((NS, CF), jnp.float32),
            pltpu.VMEM((FW * N,), jnp.int32),
            pltpu.VMEM((Y_W,), jnp.int32),
            pltpu.VMEM((Y_W,), jnp.int32),
            pltpu.VMEM((E, W), jnp.int32),
            pltpu.VMEM((E, W), jnp.int32),
            pltpu.VMEM((E, W), jnp.int32),
            pltpu.VMEM((E, W), jnp.int32),
            pltpu.VMEM((Y_W,), jnp.float32),
            pltpu.VMEM((CF,), jnp.int32),
            pltpu.VMEM((CF,), jnp.int32),
            pltpu.VMEM((CF,), jnp.int32),
            pltpu.VMEM((CF,), jnp.int32),
            pltpu.VMEM((CF,), jnp.float32),
            pltpu.VMEM((NS, RW), jnp.float32),
            pltpu.VMEM((RW,), jnp.float32),
            pltpu.VMEM((RW,), jnp.float32),
            pltpu.SemaphoreType.DMA,
            pltpu.SemaphoreType.DMA,
            pltpu.SemaphoreType.DMA,
            pltpu.SemaphoreType.DMA,
            pltpu.SemaphoreType.DMA,
            pltpu.SemaphoreType.DMA,
            pltpu.SemaphoreType.DMA,
        ],
    )(_edge_dot_kernel)
    return run(zp, zpt, src, dst)


# speedup vs baseline: 1.3369x; 1.3369x over previous
import functools

import jax
import jax.numpy as jnp
from jax import lax
from jax.experimental import pallas as pl
from jax.experimental.pallas import tpu as pltpu
from jax.experimental.pallas import tpu_sc as plsc

B = 320000
D = 128
N = 10000
W = 64
NC, NS, L = 2, 16, 16
NW = NC * NS
E_W = B // NW
E = 200
N_CHUNK = E_W // E
N_PAIR = N_CHUNK // 2
N_STAGE = N // NS


def _edge_dot_kernel(zp_hbm, src_hbm, dst_hbm, out_hbm,
                     zsh, sidx_v, didx_v,
                     sbuf_a, dbuf_a, sbuf_b, dbuf_b, out_v,
                     sem_i, sem_a, sem_b):
    c = lax.axis_index("c")
    s = lax.axis_index("s")
    wid = s * NC + c
    base_w = wid * E_W
    lane = lax.iota(jnp.int32, L)
    m15 = lane == 15

    pltpu.async_copy(src_hbm.at[pl.ds(base_w, E_W)], sidx_v, sem_i)
    pltpu.async_copy(dst_hbm.at[pl.ds(base_w, E_W)], didx_v, sem_i)
    pltpu.sync_copy(zp_hbm.at[pl.ds(s * N_STAGE, N_STAGE)],
                    zsh.at[pl.ds(s * N_STAGE, N_STAGE)])
    pltpu.make_async_copy(src_hbm.at[pl.ds(0, E_W)], sidx_v, sem_i).wait()
    pltpu.make_async_copy(src_hbm.at[pl.ds(0, E_W)], didx_v, sem_i).wait()
    plsc.subcore_barrier()

    H = 104

    def issue(k, sbuf, dbuf, sem):
        pltpu.async_copy(zsh.at[sidx_v.at[pl.ds(k * E, H)]],
                         sbuf.at[pl.ds(0, H)], sem)
        pltpu.async_copy(zsh.at[didx_v.at[pl.ds(k * E, H)]],
                         dbuf.at[pl.ds(0, H)], sem)
        pltpu.async_copy(zsh.at[sidx_v.at[pl.ds(k * E + H, E - H)]],
                         sbuf.at[pl.ds(H, E - H)], sem)
        pltpu.async_copy(zsh.at[didx_v.at[pl.ds(k * E + H, E - H)]],
                         dbuf.at[pl.ds(H, E - H)], sem)

    def wait(sbuf, dbuf, sem):
        pltpu.make_async_copy(zsh.at[pl.ds(0, H)],
                              sbuf.at[pl.ds(0, H)], sem).wait()
        pltpu.make_async_copy(zsh.at[pl.ds(0, H)],
                              dbuf.at[pl.ds(0, H)], sem).wait()
        pltpu.make_async_copy(zsh.at[pl.ds(0, E - H)],
                              sbuf.at[pl.ds(H, E - H)], sem).wait()
        pltpu.make_async_copy(zsh.at[pl.ds(0, E - H)],
                              dbuf.at[pl.ds(H, E - H)], sem).wait()

    cols = [lane + (16 * k) for k in range(W // L)]
    rowzero = jnp.zeros((L,), jnp.int32)

    def compute(k, sbuf, dbuf):
        @plsc.parallel_loop(0, E, 1, unroll=2)
        def _(e):
            base = jnp.full((L,), e * W, jnp.int32)
            acc = None
            for q in range(W // L):
                idx = base + cols[q]
                sv = plsc.bitcast(plsc.load_gather(sbuf, [rowzero, idx]),
                                  jnp.bfloat16)
                dv = plsc.bitcast(plsc.load_gather(dbuf, [rowzero, idx]),
                                  jnp.bfloat16)
                p = sv * dv
                acc = p if acc is None else acc + p
            lo, hi = plsc.unpack(acc, format=plsc.PackFormat.INTERLEAVED)
            tot = plsc.cumsum(lo.astype(jnp.float32) + hi.astype(jnp.float32))
            plsc.store_scatter(out_v, [jnp.full((L,), k * E, jnp.int32) + e],
                               tot, mask=m15)

    issue(0, sbuf_a, dbuf_a, sem_a)
    issue(1, sbuf_b, dbuf_b, sem_b)

    def pair_body(p, _):
        ka = 2 * p
        wait(sbuf_a, dbuf_a, sem_a)
        compute(ka, sbuf_a, dbuf_a)

        @pl.when(p < N_PAIR - 1)
        def _():
            issue(ka + 2, sbuf_a, dbuf_a, sem_a)

        wait(sbuf_b, dbuf_b, sem_b)

        compute(ka + 1, sbuf_b, dbuf_b)

        @pl.when(p < N_PAIR - 1)
        def _():
            issue(ka + 3, sbuf_b, dbuf_b, sem_b)

        return 0

    lax.fori_loop(0, N_PAIR, pair_body, 0)

    pltpu.sync_copy(out_v, out_hbm.at[pl.ds(base_w, E_W)])


@jax.jit
def kernel(z, edge_index):
    zp = lax.bitcast_convert_type(
        z.astype(jnp.bfloat16).reshape(N, W, 2), jnp.int32)
    src = edge_index[0].astype(jnp.int32)
    dst = edge_index[1].astype(jnp.int32)
    mesh = plsc.VectorSubcoreMesh(core_axis_name="c", subcore_axis_name="s")
    run = functools.partial(
        pl.kernel,
        mesh=mesh,
        out_type=jax.ShapeDtypeStruct((B,), jnp.float32),
        compiler_params=pltpu.CompilerParams(
            use_tc_tiling_on_sc=False, needs_layout_passes=False),
        scratch_types=[
            pltpu.VMEM_SHARED((N, W), jnp.int32),
            pltpu.VMEM((E_W,), jnp.int32),
            pltpu.VMEM((E_W,), jnp.int32),
            pltpu.VMEM((E, W), jnp.int32),
            pltpu.VMEM((E, W), jnp.int32),
            pltpu.VMEM((E, W), jnp.int32),
            pltpu.VMEM((E, W), jnp.int32),
            pltpu.VMEM((E_W,), jnp.float32),
            pltpu.SemaphoreType.DMA,
            pltpu.SemaphoreType.DMA,
            pltpu.SemaphoreType.DMA,
        ],
    )(_edge_dot_kernel)
    return run(zp, src, dst)
